# baseline (speedup 1.0000x reference)
import jax
import jax.numpy as jnp
from jax.experimental import pallas as pl
from jax.experimental.pallas import tpu as pltpu

N_CB = 9
CB_SIZE = 1024
CB_DIM = 8


def _rowsum8_lane(v):
    s04 = v[..., 0:1] + v[..., 4:5]
    s26 = v[..., 2:3] + v[..., 6:7]
    s15 = v[..., 1:2] + v[..., 5:6]
    s37 = v[..., 3:4] + v[..., 7:8]
    return (s04 + s26) + (s15 + s37)


def _colsum8_sub(v):
    s04 = v[0:1] + v[4:5]
    s26 = v[2:3] + v[6:7]
    s15 = v[1:2] + v[5:6]
    s37 = v[3:4] + v[7:8]
    return (s04 + s26) + (s15 + s37)


def _prep_kernel(cb_ref, cbn_ref, bsq_ref, hml_ref):
    cb = cb_ref[...]
    cn = jnp.sqrt(_rowsum8_lane(cb * cb))
    cb_n = cb / jnp.maximum(cn, 1e-12)
    cbn_ref[...] = cb_n
    bsq_ref[...] = _rowsum8_lane(cb_n ** 2)
    hi = cb.astype(jnp.bfloat16).astype(jnp.float32)
    r1 = cb - hi
    mid = r1.astype(jnp.bfloat16).astype(jnp.float32)
    lo = (r1 - mid).astype(jnp.bfloat16).astype(jnp.float32)
    hml_ref[...] = jnp.concatenate([hi, mid, lo], axis=2).astype(jnp.bfloat16)


def _rvq_kernel(x_ref, in_w_ref, in_b_ref, out_w_ref, out_b_ref,
                cbn_ref, bsq_ref, hml_ref, zq_ref):
    residual = x_ref[0]
    z_q = jnp.zeros_like(residual)
    tc = residual.shape[1]
    sub_iota = jax.lax.broadcasted_iota(jnp.int32, (CB_SIZE, tc), 0)
    for i in range(N_CB):
        enc = jax.lax.dot_general(
            in_w_ref[i], residual, (((1,), (0,)), ((), ())))
        enc = enc + in_b_ref[i][:, None]
        n = jnp.sqrt(_colsum8_sub(enc * enc))
        enc_n = enc / jnp.maximum(n, 1e-12)
        s = jax.lax.dot_general(
            cbn_ref[i], enc_n, (((1,), (0,)), ((), ())))
        asq = _colsum8_sub(enc_n ** 2)
        neg = (2.0 * s - asq) - bsq_ref[i]
        m = jnp.max(neg, axis=0, keepdims=True)
        cand = jnp.where(neg == m, sub_iota, CB_SIZE)
        idx = jnp.min(cand, axis=0, keepdims=True)
        onehot = (cand == idx).astype(jnp.bfloat16)
        dn = (((0,), (0,)), ((), ()))
        q24 = jax.lax.dot_general(hml_ref[i], onehot, dn,
                                  preferred_element_type=jnp.float32)
        q = (q24[0:8] + q24[8:16]) + q24[16:24]
        zqi = jax.lax.dot_general(
            out_w_ref[i], q, (((1,), (0,)), ((), ())))
        zqi = zqi + out_b_ref[i][:, None]
        z_q = z_q + zqi
        residual = residual - zqi
    zq_ref[0] = z_q


def kernel(x, in_w, in_b, out_w, out_b, codebooks):
    b, d, t = x.shape
    shp8 = jax.ShapeDtypeStruct((N_CB, CB_SIZE, CB_DIM), jnp.float32)
    cbn, bsq, hml = pl.pallas_call(
        _prep_kernel,
        out_shape=[shp8, jax.ShapeDtypeStruct((N_CB, CB_SIZE, 1), jnp.float32),
                   jax.ShapeDtypeStruct((N_CB, CB_SIZE, 3 * CB_DIM),
                                        jnp.bfloat16)],
    )(codebooks)
    tc = 1024 if t % 1024 == 0 else t
    grid = (b, t // tc)
    blk = pl.BlockSpec((1, d, tc), lambda i, j: (i, 0, j))
    full = lambda a: pl.BlockSpec(a.shape, lambda i, j: (0,) * a.ndim)
    return pl.pallas_call(
        _rvq_kernel,
        grid=grid,
        in_specs=[blk, full(in_w), full(in_b), full(out_w), full(out_b),
                  full(cbn), full(bsq), full(hml)],
        out_specs=blk,
        out_shape=jax.ShapeDtypeStruct((b, d, t), x.dtype),
        compiler_params=pltpu.CompilerParams(
            dimension_semantics=("parallel", "parallel")),
    )(x, in_w, in_b, out_w, out_b, cbn, bsq, hml)


# speedup vs baseline: 1.0763x; 1.0763x over previous
import jax
import jax.numpy as jnp
from jax.experimental import pallas as pl
from jax.experimental.pallas import tpu as pltpu

N_CB = 9
CB_SIZE = 1024
CB_DIM = 8


def _rowsum8_lane(v):
    s04 = v[..., 0:1] + v[..., 4:5]
    s26 = v[..., 2:3] + v[..., 6:7]
    s15 = v[..., 1:2] + v[..., 5:6]
    s37 = v[..., 3:4] + v[..., 7:8]
    return (s04 + s26) + (s15 + s37)


def _colsum8_sub(v):
    s04 = v[0:1] + v[4:5]
    s26 = v[2:3] + v[6:7]
    s15 = v[1:2] + v[5:6]
    s37 = v[3:4] + v[7:8]
    return (s04 + s26) + (s15 + s37)


def _prep_kernel(cb_ref, cbn_ref, bsq_ref, hml_ref):
    cb = cb_ref[...]
    cn = jnp.sqrt(_rowsum8_lane(cb * cb))
    cb_n = cb / jnp.maximum(cn, 1e-12)
    cbn_ref[...] = cb_n
    bsq_ref[...] = _rowsum8_lane(cb_n ** 2)
    hi = cb.astype(jnp.bfloat16).astype(jnp.float32)
    r1 = cb - hi
    mid = r1.astype(jnp.bfloat16).astype(jnp.float32)
    lo = (r1 - mid).astype(jnp.bfloat16).astype(jnp.float32)
    hml_ref[...] = jnp.concatenate([hi, mid, lo], axis=2).astype(jnp.bfloat16)


def _rvq_kernel(x_ref, in_w_ref, in_b_ref, out_w_ref, out_b_ref,
                cbn_ref, bsq_ref, hml_ref, zq_ref):
    residual = x_ref[0]
    z_q = jnp.zeros_like(residual)
    tc = residual.shape[1]
    sub_iota = jax.lax.broadcasted_iota(jnp.int32, (CB_SIZE, tc), 0)
    for i in range(N_CB):
        enc = jax.lax.dot_general(
            in_w_ref[i], residual, (((1,), (0,)), ((), ())))
        enc = enc + in_b_ref[i][:, None]
        n = jnp.sqrt(_colsum8_sub(enc * enc))
        enc_n = enc / jnp.maximum(n, 1e-12)
        s = jax.lax.dot_general(
            cbn_ref[i], enc_n, (((1,), (0,)), ((), ())))
        asq = _colsum8_sub(enc_n ** 2)
        neg = (2.0 * s - asq) - bsq_ref[i]
        m = jnp.max(neg, axis=0, keepdims=True)
        idx = jnp.min(jnp.where(neg == m, sub_iota, CB_SIZE),
                      axis=0, keepdims=True)
        onehot = (sub_iota == idx).astype(jnp.bfloat16)
        dn = (((0,), (0,)), ((), ()))
        q24 = jax.lax.dot_general(hml_ref[i], onehot, dn,
                                  preferred_element_type=jnp.float32)
        q = (q24[0:8] + q24[8:16]) + q24[16:24]
        zqi = jax.lax.dot_general(
            out_w_ref[i], q, (((1,), (0,)), ((), ())))
        zqi = zqi + out_b_ref[i][:, None]
        z_q = z_q + zqi
        residual = residual - zqi
    zq_ref[0] = z_q


def kernel(x, in_w, in_b, out_w, out_b, codebooks):
    b, d, t = x.shape
    shp8 = jax.ShapeDtypeStruct((N_CB, CB_SIZE, CB_DIM), jnp.float32)
    cbn, bsq, hml = pl.pallas_call(
        _prep_kernel,
        out_shape=[shp8, jax.ShapeDtypeStruct((N_CB, CB_SIZE, 1), jnp.float32),
                   jax.ShapeDtypeStruct((N_CB, CB_SIZE, 3 * CB_DIM),
                                        jnp.bfloat16)],
    )(codebooks)
    tc = 1024 if t % 1024 == 0 else t
    grid = (b, t // tc)
    blk = pl.BlockSpec((1, d, tc), lambda i, j: (i, 0, j))
    full = lambda a: pl.BlockSpec(a.shape, lambda i, j: (0,) * a.ndim)
    return pl.pallas_call(
        _rvq_kernel,
        grid=grid,
        in_specs=[blk, full(in_w), full(in_b), full(out_w), full(out_b),
                  full(cbn), full(bsq), full(hml)],
        out_specs=blk,
        out_shape=jax.ShapeDtypeStruct((b, d, t), x.dtype),
        compiler_params=pltpu.CompilerParams(
            dimension_semantics=("parallel", "parallel")),
    )(x, in_w, in_b, out_w, out_b, cbn, bsq, hml)


# speedup vs baseline: 1.0789x; 1.0024x over previous
import jax
import jax.numpy as jnp
from jax.experimental import pallas as pl
from jax.experimental.pallas import tpu as pltpu

N_CB = 9
CB_SIZE = 1024
CB_DIM = 8


def _rowsum8_lane(v):
    s04 = v[..., 0:1] + v[..., 4:5]
    s26 = v[..., 2:3] + v[..., 6:7]
    s15 = v[..., 1:2] + v[..., 5:6]
    s37 = v[..., 3:4] + v[..., 7:8]
    return (s04 + s26) + (s15 + s37)


def _colsum8_sub(v):
    s04 = v[0:1] + v[4:5]
    s26 = v[2:3] + v[6:7]
    s15 = v[1:2] + v[5:6]
    s37 = v[3:4] + v[7:8]
    return (s04 + s26) + (s15 + s37)


def _prep_kernel(cb_ref, cbn_ref, bsq_ref, hml_ref):
    cb = cb_ref[...]
    cn = jnp.sqrt(_rowsum8_lane(cb * cb))
    cb_n = cb / jnp.maximum(cn, 1e-12)
    cbn_ref[...] = cb_n
    bsq_ref[...] = _rowsum8_lane(cb_n ** 2)
    hi = cb.astype(jnp.bfloat16).astype(jnp.float32)
    r1 = cb - hi
    mid = r1.astype(jnp.bfloat16).astype(jnp.float32)
    lo = (r1 - mid).astype(jnp.bfloat16).astype(jnp.float32)
    hml_ref[...] = jnp.concatenate([hi, mid, lo], axis=2).astype(jnp.bfloat16)


def _rvq_kernel(x_ref, in_w_ref, in_b_ref, out_w_ref, out_b_ref,
                cbn_ref, bsq_ref, hml_ref, zq_ref):
    residual = x_ref[0]
    z_q = jnp.zeros_like(residual)
    tc = residual.shape[1]
    sub_iota = jax.lax.broadcasted_iota(jnp.int32, (CB_SIZE, 1), 0)
    for i in range(N_CB):
        enc = jax.lax.dot_general(
            in_w_ref[i], residual, (((1,), (0,)), ((), ())))
        enc = enc + in_b_ref[i][:, None]
        n = jnp.sqrt(_colsum8_sub(enc * enc))
        enc_n = enc / jnp.maximum(n, 1e-12)
        s = jax.lax.dot_general(
            cbn_ref[i], enc_n, (((1,), (0,)), ((), ())))
        asq = _colsum8_sub(enc_n ** 2)
        neg = (2.0 * s - asq) - bsq_ref[i]
        m = jnp.max(neg, axis=0, keepdims=True)
        idx = jnp.min(jnp.where(neg == m, sub_iota, CB_SIZE),
                      axis=0, keepdims=True)
        onehot = (sub_iota == idx).astype(jnp.bfloat16)
        dn = (((0,), (0,)), ((), ()))
        q24 = jax.lax.dot_general(hml_ref[i], onehot, dn,
                                  preferred_element_type=jnp.float32)
        q = (q24[0:8] + q24[8:16]) + q24[16:24]
        zqi = jax.lax.dot_general(
            out_w_ref[i], q, (((1,), (0,)), ((), ())))
        zqi = zqi + out_b_ref[i][:, None]
        z_q = z_q + zqi
        residual = residual - zqi
    zq_ref[0] = z_q


def kernel(x, in_w, in_b, out_w, out_b, codebooks):
    b, d, t = x.shape
    shp8 = jax.ShapeDtypeStruct((N_CB, CB_SIZE, CB_DIM), jnp.float32)
    cbn, bsq, hml = pl.pallas_call(
        _prep_kernel,
        out_shape=[shp8, jax.ShapeDtypeStruct((N_CB, CB_SIZE, 1), jnp.float32),
                   jax.ShapeDtypeStruct((N_CB, CB_SIZE, 3 * CB_DIM),
                                        jnp.bfloat16)],
    )(codebooks)
    tc = 1024 if t % 1024 == 0 else t
    grid = (b, t // tc)
    blk = pl.BlockSpec((1, d, tc), lambda i, j: (i, 0, j))
    full = lambda a: pl.BlockSpec(a.shape, lambda i, j: (0,) * a.ndim)
    return pl.pallas_call(
        _rvq_kernel,
        grid=grid,
        in_specs=[blk, full(in_w), full(in_b), full(out_w), full(out_b),
                  full(cbn), full(bsq), full(hml)],
        out_specs=blk,
        out_shape=jax.ShapeDtypeStruct((b, d, t), x.dtype),
        compiler_params=pltpu.CompilerParams(
            dimension_semantics=("parallel", "parallel")),
    )(x, in_w, in_b, out_w, out_b, cbn, bsq, hml)
